# baseline (speedup 1.0000x reference)
import jax
import jax.numpy as jnp
from jax.experimental import pallas as pl
from jax.experimental.pallas import tpu as pltpu

_DIM = 4096


def _chain(t_iters):
    def k(a_ref, o_ref):
        def body(t, x):
            return jnp.dot(x, a_ref[...], preferred_element_type=jnp.float32)
        o_ref[0] = jax.lax.fori_loop(0, t_iters, body, a_ref[...])
    return k


def kernel(eye, jacobian):
    a = eye.reshape(_DIM * _DIM // 128, 128)[0:8192].reshape(1024, 1024)
    pa = pl.pallas_call(
        _chain(64), grid=(1,),
        in_specs=[pl.BlockSpec((1024, 1024), lambda i: (0, 0))],
        out_specs=pl.BlockSpec((1, 1024, 1024), lambda i: (i, 0, 0)),
        out_shape=jax.ShapeDtypeStruct((1, 1024, 1024), jnp.float32),
        compiler_params=pltpu.CompilerParams(
            dimension_semantics=("parallel",)),
    )(a)
    pb = pl.pallas_call(
        _chain(32), grid=(2,),
        in_specs=[pl.BlockSpec((1024, 1024), lambda i: (0, 0))],
        out_specs=pl.BlockSpec((1, 1024, 1024), lambda i: (i, 0, 0)),
        out_shape=jax.ShapeDtypeStruct((2, 1024, 1024), jnp.float32),
        compiler_params=pltpu.CompilerParams(
            dimension_semantics=("parallel",)),
    )(a)
    acc = pa[0, 0, 0] + pb[0, 0, 0] + pb[1, 0, 0]
    return jnp.full((_DIM * _DIM,), acc, jnp.float32)


# speedup vs baseline: 2.3563x; 2.3563x over previous
import jax
import jax.numpy as jnp
from jax.experimental import pallas as pl
from jax.experimental.pallas import tpu as pltpu

_DIM = 4096
_BM = 1024
_NI = _DIM // _BM
_NS = 8
_NV = _DIM // 128
_NC = 4


def _lhs_copy(j_hbm, lhsb, lsems, lbuf, i, v):
    return pltpu.make_async_copy(
        j_hbm.at[pl.ds(i * _BM, _BM), v, :],
        lhsb.at[lbuf, :, pl.ds(128 * v, 128)],
        lsems.at[lbuf, v])


def _rhs_copy(e_hbm, rhsb, rsems, buf, s, c):
    return pltpu.make_async_copy(
        e_hbm.at[:, _NC * s + c, :],
        rhsb.at[buf, :, pl.ds(128 * c, 128)],
        rsems.at[buf, c])


def _out_copy(o_hbm, outb, osems, buf, i, s, c):
    return pltpu.make_async_copy(
        outb.at[buf, :, pl.ds(128 * c, 128)],
        o_hbm.at[pl.ds(i * _BM, _BM), _NC * s + c, :],
        osems.at[buf, c])


def _mm_kernel(j_hbm, e_hbm, o_hbm, lhsb, rhsb, outb, lsems, rsems, osems):
    g = pl.program_id(0)
    i = g // _NS
    s = jax.lax.rem(g, _NS)
    buf = jax.lax.rem(g, 2)
    lbuf = jax.lax.rem(i, 2)

    @pl.when(g == 0)
    def _start_first():
        for c in range(_NC):
            _rhs_copy(e_hbm, rhsb, rsems, 0, 0, c).start()
        for v in range(_NV):
            _lhs_copy(j_hbm, lhsb, lsems, 0, 0, v).start()

    @pl.when(g + 1 < _NI * _NS)
    def _prefetch_rhs():
        sn = jax.lax.rem(g + 1, _NS)
        for c in range(_NC):
            _rhs_copy(e_hbm, rhsb, rsems, 1 - buf, sn, c).start()

    @pl.when(jnp.logical_and(s == 2, i + 1 < _NI))
    def _prefetch_lhs():
        for v in range(_NV):
            _lhs_copy(j_hbm, lhsb, lsems, 1 - lbuf, i + 1, v).start()

    @pl.when(s == 0)
    def _wait_lhs():
        for v in range(_NV):
            _lhs_copy(j_hbm, lhsb, lsems, lbuf, i, v).wait()

    for c in range(_NC):
        _rhs_copy(e_hbm, rhsb, rsems, buf, s, c).wait()

    @pl.when(g >= 2)
    def _wait_prev_out():
        g2 = g - 2
        for c in range(_NC):
            _out_copy(o_hbm, outb, osems, buf, g2 // _NS,
                      jax.lax.rem(g2, _NS), c).wait()

    outb[buf] = jnp.dot(lhsb[lbuf], rhsb[buf],
                        preferred_element_type=jnp.float32)

    for c in range(_NC):
        _out_copy(o_hbm, outb, osems, buf, i, s, c).start()

    @pl.when(g == _NI * _NS - 1)
    def _drain():
        g1 = g - 1
        for c in range(_NC):
            _out_copy(o_hbm, outb, osems, 1 - buf, g1 // _NS,
                      jax.lax.rem(g1, _NS), c).wait()
            _out_copy(o_hbm, outb, osems, buf, i, s, c).wait()


def kernel(eye, jacobian):
    j3 = jacobian.reshape(_DIM, _NV, 128)
    e3 = eye.reshape(_DIM, _NV, 128)
    out = pl.pallas_call(
        _mm_kernel,
        grid=(_NI * _NS,),
        in_specs=[
            pl.BlockSpec(memory_space=pl.ANY),
            pl.BlockSpec(memory_space=pl.ANY),
        ],
        out_specs=pl.BlockSpec(memory_space=pl.ANY),
        out_shape=jax.ShapeDtypeStruct((_DIM, _NV, 128), jnp.float32),
        scratch_shapes=[
            pltpu.VMEM((2, _BM, _DIM), jnp.float32),
            pltpu.VMEM((2, _DIM, 128 * _NC), jnp.float32),
            pltpu.VMEM((2, _BM, 128 * _NC), jnp.float32),
            pltpu.SemaphoreType.DMA((2, _NV)),
            pltpu.SemaphoreType.DMA((2, _NC)),
            pltpu.SemaphoreType.DMA((2, _NC)),
        ],
        compiler_params=pltpu.CompilerParams(
            dimension_semantics=("arbitrary",),
            vmem_limit_bytes=56 * 1024 * 1024,
        ),
    )(j3, e3)
    return out.reshape(_DIM * _DIM)


# speedup vs baseline: 2.4622x; 1.0450x over previous
import jax
import jax.numpy as jnp
from jax.experimental import pallas as pl
from jax.experimental.pallas import tpu as pltpu

_DIM = 4096
_BM = 1024
_NI = _DIM // _BM
_NS = 8
_NV = _DIM // 128
_NC = 4


def _lhs_copy(j_hbm, lhsb, lsems, lbuf, i, v):
    return pltpu.make_async_copy(
        j_hbm.at[pl.ds(i * _BM, _BM), v, :],
        lhsb.at[lbuf, :, pl.ds(128 * v, 128)],
        lsems.at[lbuf, v])


def _rhs_copy(e_hbm, rhsb, rsems, buf, s, c):
    return pltpu.make_async_copy(
        e_hbm.at[:, _NC * s + c, :],
        rhsb.at[buf, :, pl.ds(128 * c, 128)],
        rsems.at[buf, c])


def _out_copy(o_hbm, outb, osems, buf, i, s, c):
    return pltpu.make_async_copy(
        outb.at[buf, :, pl.ds(128 * c, 128)],
        o_hbm.at[pl.ds(i * _BM, _BM), _NC * s + c, :],
        osems.at[buf, c])


def _mm_kernel(j_hbm, e_hbm, o_hbm, lhsb, rhsb, outb, lsems, rsems, osems):
    g = pl.program_id(0)
    i = g // _NS
    s = jax.lax.rem(g, _NS)
    buf = jax.lax.rem(g, 2)
    lbuf = jax.lax.rem(i, 2)

    @pl.when(g == 0)
    def _start_first():
        for c in range(_NC):
            _rhs_copy(e_hbm, rhsb, rsems, 0, 0, c).start()
        for v in range(_NV):
            _lhs_copy(j_hbm, lhsb, lsems, 0, 0, v).start()

    @pl.when(g + 1 < _NI * _NS)
    def _prefetch_rhs():
        sn = jax.lax.rem(g + 1, _NS)
        for c in range(_NC):
            _rhs_copy(e_hbm, rhsb, rsems, 1 - buf, sn, c).start()

    for sp in range(2, 6):
        @pl.when(jnp.logical_and(s == sp, i + 1 < _NI))
        def _prefetch_lhs(sp=sp):
            for v in range(8 * (sp - 2), 8 * (sp - 1)):
                _lhs_copy(j_hbm, lhsb, lsems, 1 - lbuf, i + 1, v).start()

    @pl.when(s == 0)
    def _wait_lhs():
        for v in range(_NV):
            _lhs_copy(j_hbm, lhsb, lsems, lbuf, i, v).wait()

    for c in range(_NC):
        _rhs_copy(e_hbm, rhsb, rsems, buf, s, c).wait()

    @pl.when(g >= 2)
    def _wait_prev_out():
        g2 = g - 2
        for c in range(_NC):
            _out_copy(o_hbm, outb, osems, buf, g2 // _NS,
                      jax.lax.rem(g2, _NS), c).wait()

    outb[buf] = jnp.dot(lhsb[lbuf], rhsb[buf],
                        preferred_element_type=jnp.float32)

    for c in range(_NC):
        _out_copy(o_hbm, outb, osems, buf, i, s, c).start()

    @pl.when(g == _NI * _NS - 1)
    def _drain():
        g1 = g - 1
        for c in range(_NC):
            _out_copy(o_hbm, outb, osems, 1 - buf, g1 // _NS,
                      jax.lax.rem(g1, _NS), c).wait()
            _out_copy(o_hbm, outb, osems, buf, i, s, c).wait()


def kernel(eye, jacobian):
    j3 = jacobian.reshape(_DIM, _NV, 128)
    e3 = eye.reshape(_DIM, _NV, 128)
    out = pl.pallas_call(
        _mm_kernel,
        grid=(_NI * _NS,),
        in_specs=[
            pl.BlockSpec(memory_space=pl.ANY),
            pl.BlockSpec(memory_space=pl.ANY),
        ],
        out_specs=pl.BlockSpec(memory_space=pl.ANY),
        out_shape=jax.ShapeDtypeStruct((_DIM, _NV, 128), jnp.float32),
        scratch_shapes=[
            pltpu.VMEM((2, _BM, _DIM), jnp.float32),
            pltpu.VMEM((2, _DIM, 128 * _NC), jnp.float32),
            pltpu.VMEM((2, _BM, 128 * _NC), jnp.float32),
            pltpu.SemaphoreType.DMA((2, _NV)),
            pltpu.SemaphoreType.DMA((2, _NC)),
            pltpu.SemaphoreType.DMA((2, _NC)),
        ],
        compiler_params=pltpu.CompilerParams(
            dimension_semantics=("arbitrary",),
            vmem_limit_bytes=56 * 1024 * 1024,
        ),
    )(j3, e3)
    return out.reshape(_DIM * _DIM)
